# baseline (speedup 1.0000x reference)
import functools

import jax
import jax.numpy as jnp
from jax import lax
from jax.experimental import pallas as pl
from jax.experimental.pallas import tpu as pltpu
from jax.experimental.pallas import tpu_sc as plsc

BATCH = 16384
NF = 64

_info = plsc.get_sparse_core_info()
_NC, _NS = _info.num_cores, _info.num_subcores
_NW = _NC * _NS
_BPW = BATCH // _NW


def _repack_body(lo_ref, hi_ref, out_ref):
    out_ref[:, :NF] = lo_ref[:]
    out_ref[:, NF:] = hi_ref[:]


def _repack(table, block_rows):
    n = table.shape[0]
    half = n // 2
    assert half % block_rows == 0
    grid = half // block_rows
    return pl.pallas_call(
        _repack_body,
        grid=(grid,),
        in_specs=[
            pl.BlockSpec((block_rows, NF), lambda i: (i, 0)),
            pl.BlockSpec((block_rows, NF), lambda i, g=grid: (i + g, 0)),
        ],
        out_specs=pl.BlockSpec((block_rows, 2 * NF), lambda i: (i, 0)),
        out_shape=jax.ShapeDtypeStruct((half, 2 * NF), jnp.float32),
    )(table, table)


def _gather_body(Up_hbm, Mp_hbm, users_hbm, movies_hbm, eu_hbm, em_hbm,
                 idx_v, tix_v, rows_v, sem):
    wid = lax.axis_index("s") * _NC + lax.axis_index("c")
    base = wid * _BPW

    def do_table(tab_hbm, src_hbm, out_hbm, half):
        pltpu.sync_copy(src_hbm.at[pl.ds(base, _BPW)], idx_v)
        for k in range(_BPW // 16):
            iv = idx_v[pl.ds(k * 16, 16)]
            wrap = jnp.where(iv >= half, half, 0)
            tix_v[pl.ds(k * 16, 16)] = iv - wrap
        pltpu.async_copy(tab_hbm.at[tix_v], rows_v, sem).wait()
        pltpu.sync_copy(rows_v, out_hbm.at[pl.ds(base, _BPW)])

    do_table(Up_hbm, users_hbm, eu_hbm, 500000)
    do_table(Mp_hbm, movies_hbm, em_hbm, 50000)


_sc_gather = functools.partial(
    pl.kernel,
    out_type=(
        jax.ShapeDtypeStruct((BATCH, 2 * NF), jnp.float32),
        jax.ShapeDtypeStruct((BATCH, 2 * NF), jnp.float32),
    ),
    mesh=plsc.VectorSubcoreMesh(core_axis_name="c", subcore_axis_name="s"),
    scratch_types=[
        pltpu.VMEM((_BPW,), jnp.int32),
        pltpu.VMEM((_BPW,), jnp.int32),
        pltpu.VMEM((_BPW, 2 * NF), jnp.float32),
        pltpu.SemaphoreType.DMA,
    ],
)(_gather_body)


def _mlp_body(eu_ref, em_ref, pu_ref, pm_ref, w1u_ref, w1m_ref, b1_ref,
              w2_ref, b2_ref, out_ref):
    eu = eu_ref[:]
    em = em_ref[:]
    au = jnp.dot(eu[:, :NF], w1u_ref[:], preferred_element_type=jnp.float32)
    bu = jnp.dot(eu[:, NF:], w1u_ref[:], preferred_element_type=jnp.float32)
    am = jnp.dot(em[:, :NF], w1m_ref[:], preferred_element_type=jnp.float32)
    bm = jnp.dot(em[:, NF:], w1m_ref[:], preferred_element_type=jnp.float32)
    hu = jnp.where(pu_ref[:] > 0, bu, au)
    hm = jnp.where(pm_ref[:] > 0, bm, am)
    h = jnp.maximum(hu + hm + b1_ref[:], 0.0)
    o = jnp.dot(h, w2_ref[:], preferred_element_type=jnp.float32) + b2_ref[:]
    out_ref[:] = jax.nn.sigmoid(o) * 6.0 - 0.5


def kernel(users, movies, U, M, W1, b1, W2, b2):
    users = users.astype(jnp.int32)
    movies = movies.astype(jnp.int32)
    Up = _repack(U, 2000)
    Mp = _repack(M, 1000)
    eu, em = _sc_gather(Up, Mp, users, movies)
    pu = (users >= 500000)[:, None].astype(jnp.int32)
    pm = (movies >= 50000)[:, None].astype(jnp.int32)
    w1u = W1[:, :NF].T
    w1m = W1[:, NF:].T
    blk = 4096
    out2d = pl.pallas_call(
        _mlp_body,
        grid=(BATCH // blk,),
        in_specs=[
            pl.BlockSpec((blk, 2 * NF), lambda i: (i, 0)),
            pl.BlockSpec((blk, 2 * NF), lambda i: (i, 0)),
            pl.BlockSpec((blk, 1), lambda i: (i, 0)),
            pl.BlockSpec((blk, 1), lambda i: (i, 0)),
            pl.BlockSpec((NF, 10), lambda i: (0, 0)),
            pl.BlockSpec((NF, 10), lambda i: (0, 0)),
            pl.BlockSpec((1, 10), lambda i: (0, 0)),
            pl.BlockSpec((10, 1), lambda i: (0, 0)),
            pl.BlockSpec((1, 1), lambda i: (0, 0)),
        ],
        out_specs=pl.BlockSpec((blk, 1), lambda i: (i, 0)),
        out_shape=jax.ShapeDtypeStruct((BATCH, 1), jnp.float32),
    )(eu, em, pu, pm, w1u, w1m, b1[None, :], W2.T, b2[None, :])
    return out2d[:, 0]
